# baseline (speedup 1.0000x reference)
import functools
import jax
import jax.numpy as jnp
from jax import lax
from jax.experimental import pallas as pl
from jax.experimental.pallas import tpu as pltpu
from jax.experimental.pallas import tpu_sc as plsc

NUM_TYPES = 119
N_NODES = 100000
OUT_COLS = NUM_TYPES + 1

B = 4096
GRID = -(-N_NODES // B)
NP = GRID * B


def _tc_tail_body(w, t_ref, x_ref, y_ref, z_ref, f_in_ref, a_ref):
    del f_in_ref
    _tc_body_w(w, t_ref, x_ref, y_ref, z_ref, a_ref)


def _tc_body_w(w, t_ref, x_ref, y_ref, z_ref, a_ref):
    t = t_ref[0]
    cls = jax.lax.broadcasted_iota(jnp.int32, (OUT_COLS, w), 0)
    one_hot = (cls == t).astype(jnp.float32)
    x = x_ref[0]
    y = y_ref[0]
    z = z_ref[0]
    s = x * x + y * y + z * z
    norm = jnp.sqrt(s)
    d = jnp.maximum(norm, 1e-12)
    sn = s / (d * d)
    a_ref[:] = jnp.where(cls == NUM_TYPES, sn, one_hot)


W = 128
N_FULL = N_NODES // W
TAIL = N_NODES - N_FULL * W
NC, NS = 2, 16
NW = NC * NS
K_MAX = -(-N_FULL // NW)
GW = W // 16


def _sc_body(t_hbm, x_hbm, y_hbm, z_hbm, zz_hbm, out_hbm, tb, xb, yb, zb, buf):
    wid = lax.axis_index("s") * NC + lax.axis_index("c")
    lane = lax.iota(jnp.int32, 16)
    ones = jnp.full((16,), 1.0, jnp.float32)
    zeros = jnp.zeros((16,), jnp.float32)
    c119 = jnp.full((16,), NUM_TYPES, jnp.int32)

    pltpu.sync_copy(zz_hbm, buf)

    def _groups(n_groups, store_val_fn):
        for g in range(n_groups):
            cols = g * 16 + lane
            t = tb[pl.ds(g * 16, 16)]
            x = xb[pl.ds(g * 16, 16)]
            y = yb[pl.ds(g * 16, 16)]
            z = zb[pl.ds(g * 16, 16)]
            s = x * x + y * y + z * z
            sn = s / jnp.maximum(s, 1e-24)
            plsc.store_scatter(buf, [t, cols], ones)
            plsc.store_scatter(buf, [c119, cols], sn)

    def _clear(n_groups):
        for g in range(n_groups):
            cols = g * 16 + lane
            t = tb[pl.ds(g * 16, 16)]
            plsc.store_scatter(buf, [t, cols], zeros)

    def _load(base, n):
        pltpu.sync_copy(t_hbm.at[pl.ds(base, n)], tb.at[pl.ds(0, n)])
        pltpu.sync_copy(x_hbm.at[pl.ds(base, n)], xb.at[pl.ds(0, n)])
        pltpu.sync_copy(y_hbm.at[pl.ds(base, n)], yb.at[pl.ds(0, n)])
        pltpu.sync_copy(z_hbm.at[pl.ds(base, n)], zb.at[pl.ds(0, n)])

    def step(k, carry):
        c = wid + NW * k

        @pl.when(c < N_FULL)
        def _():
            base = c * W
            _load(base, W)
            _groups(GW, None)
            pltpu.sync_copy(buf, out_hbm.at[:, pl.ds(base, W)])
            _clear(GW)

        return carry

    lax.fori_loop(0, K_MAX, step, 0)


@functools.cache
def _sc_call():
    return pl.kernel(
        _sc_body,
        out_type=jax.ShapeDtypeStruct((OUT_COLS, N_NODES), jnp.float32),
        mesh=plsc.VectorSubcoreMesh(
            core_axis_name="c", subcore_axis_name="s", num_cores=NC, num_subcores=NS
        ),
        scratch_types=[
            pltpu.VMEM((W,), jnp.int32),
            pltpu.VMEM((W,), jnp.float32),
            pltpu.VMEM((W,), jnp.float32),
            pltpu.VMEM((W,), jnp.float32),
            pltpu.VMEM((OUT_COLS, W), jnp.float32),
        ],
        compiler_params=pltpu.CompilerParams(needs_layout_passes=False),
    )


def kernel(atom_type, pos, spin):
    del pos
    t_flat = atom_type.reshape(N_NODES)
    sx = spin[:, 0]
    sy = spin[:, 1]
    sz = spin[:, 2]

    pad = (0, NP - N_NODES)
    t3 = jnp.pad(t_flat, pad).reshape(GRID, 1, B)
    x3 = jnp.pad(sx, pad).reshape(GRID, 1, B)
    y3 = jnp.pad(sy, pad).reshape(GRID, 1, B)
    z3 = jnp.pad(sz, pad).reshape(GRID, 1, B)
    in_spec = pl.BlockSpec((1, 1, B), lambda i: (i, 0, 0))
    out_spec = pl.BlockSpec((OUT_COLS, B), lambda i: (0, i))
    import functools as _ft
    attrs_t = pl.pallas_call(
        _ft.partial(_tc_body_w, B),
        grid=(GRID,),
        in_specs=[in_spec, in_spec, in_spec, in_spec],
        out_specs=out_spec,
        out_shape=jax.ShapeDtypeStruct((OUT_COLS, N_NODES), jnp.float32),
    )(t3, x3, y3, z3)

    zeros_chunk = jnp.zeros((OUT_COLS, W), jnp.float32)
    feats_t = _sc_call()(t_flat, sx, sy, sz, zeros_chunk)

    base = N_FULL * W
    tpad = (0, W - TAIL)
    t_t3 = jnp.pad(t_flat[base:], tpad).reshape(1, 1, W)
    x_t3 = jnp.pad(sx[base:], tpad).reshape(1, 1, W)
    y_t3 = jnp.pad(sy[base:], tpad).reshape(1, 1, W)
    z_t3 = jnp.pad(sz[base:], tpad).reshape(1, 1, W)
    tail_in = pl.BlockSpec((1, 1, W), lambda i: (0, 0, 0))
    feats_t = pl.pallas_call(
        _ft.partial(_tc_tail_body, W),
        grid=(1,),
        in_specs=[tail_in, tail_in, tail_in, tail_in, pl.BlockSpec((OUT_COLS, W), lambda i: (0, N_FULL))],
        out_specs=pl.BlockSpec((OUT_COLS, W), lambda i: (0, N_FULL)),
        out_shape=jax.ShapeDtypeStruct((OUT_COLS, N_NODES), jnp.float32),
        input_output_aliases={4: 0},
    )(t_t3, x_t3, y_t3, z_t3, feats_t)
    return (attrs_t.T, feats_t.T, spin)


# speedup vs baseline: 2.5655x; 2.5655x over previous
import jax
import jax.numpy as jnp
from jax.experimental import pallas as pl

NUM_TYPES = 119
N_NODES = 100000
OUT_COLS = NUM_TYPES + 1

B = 8192
GRID = -(-N_NODES // B)
NP = GRID * B


def _body(t_ref, x_ref, y_ref, z_ref, a_ref, f_ref):
    t = t_ref[0]
    cls = jax.lax.broadcasted_iota(jnp.int32, (OUT_COLS, B), 0)
    one_hot = (cls == t).astype(jnp.float32)
    x = x_ref[0]
    y = y_ref[0]
    z = z_ref[0]
    s = x * x + y * y + z * z
    norm = jnp.sqrt(s)
    d = jnp.maximum(norm, 1e-12)
    sn = s / (d * d)
    out = jnp.where(cls == NUM_TYPES, sn, one_hot)
    a_ref[:] = out
    f_ref[:] = out


def kernel(atom_type, pos, spin):
    del pos
    pad = (0, NP - N_NODES)
    t3 = jnp.pad(atom_type.reshape(N_NODES), pad).reshape(GRID, 1, B)
    x3 = jnp.pad(spin[:, 0], pad).reshape(GRID, 1, B)
    y3 = jnp.pad(spin[:, 1], pad).reshape(GRID, 1, B)
    z3 = jnp.pad(spin[:, 2], pad).reshape(GRID, 1, B)
    in_spec = pl.BlockSpec((1, 1, B), lambda i: (i, 0, 0))
    out_spec = pl.BlockSpec((OUT_COLS, B), lambda i: (0, i))
    out_t = jax.ShapeDtypeStruct((OUT_COLS, N_NODES), jnp.float32)
    attrs_t, feats_t = pl.pallas_call(
        _body,
        grid=(GRID,),
        in_specs=[in_spec, in_spec, in_spec, in_spec],
        out_specs=[out_spec, out_spec],
        out_shape=[out_t, out_t],
    )(t3, x3, y3, z3)
    return (attrs_t.T, feats_t.T, spin)
